# Initial kernel scaffold; baseline (speedup 1.0000x reference)
import jax
import jax.numpy as jnp
from jax.experimental import pallas as pl


def kernel(x, edge_index, edge_attr, Ws1, Wd1, We1, as1, ad1, ae1, b1, Ws2, Wd2, We2, as2, ad2, ae2, b2):
    raise NotImplementedError("write your pallas kernel here")


# speedup vs baseline: 12.1660x; 12.1660x over previous
import functools

import jax
import jax.numpy as jnp
from jax import lax
from jax.experimental import pallas as pl
from jax.experimental.pallas import tpu as pltpu
from jax.experimental.pallas import tpu_sc as plsc

NC = 2
NS = 16
LANES = 16
CH = 128


def _dense_body(x_ref, Ws_ref, Wd_ref, as_ref, ad_ref,
                xs_ref, asrc_ref, adst_ref, mxs_ref, mxd_ref):
    i = pl.program_id(0)
    x = x_ref[...]
    xs = jnp.dot(x, Ws_ref[...], preferred_element_type=jnp.float32)
    xd = jnp.dot(x, Wd_ref[...], preferred_element_type=jnp.float32)
    xs_ref[...] = xs
    asrc = jnp.sum(xs * as_ref[...], axis=1, keepdims=True)
    adst = jnp.sum(xd * ad_ref[...], axis=1, keepdims=True)
    asrc_ref[...] = asrc
    adst_ref[...] = adst
    bs = jnp.max(asrc)
    bd = jnp.max(adst)

    @pl.when(i == 0)
    def _():
        mxs_ref[0, 0] = bs
        mxd_ref[0, 0] = bd

    @pl.when(i > 0)
    def _():
        mxs_ref[0, 0] = jnp.maximum(mxs_ref[0, 0], bs)
        mxd_ref[0, 0] = jnp.maximum(mxd_ref[0, 0], bd)


def _fused_body(acc_ref, den_ref, b_ref, Ws_ref, Wd_ref, as_ref, ad_ref,
                xs_ref, asrc_ref, adst_ref, mxs_ref, mxd_ref):
    i = pl.program_id(0)
    acc = acc_ref[0] + acc_ref[1]
    den = den_ref[0, :, 0:1] + den_ref[1, :, 0:1]
    h = acc / (den + 1e-16) + b_ref[...]
    h = jnp.maximum(h, 0.0)
    xs = jnp.dot(h, Ws_ref[...], preferred_element_type=jnp.float32)
    xd = jnp.dot(h, Wd_ref[...], preferred_element_type=jnp.float32)
    xs_ref[...] = xs
    asrc = jnp.sum(xs * as_ref[...], axis=1, keepdims=True)
    adst = jnp.sum(xd * ad_ref[...], axis=1, keepdims=True)
    asrc_ref[...] = asrc
    adst_ref[...] = adst
    bs = jnp.max(asrc)
    bd = jnp.max(adst)

    @pl.when(i == 0)
    def _():
        mxs_ref[0, 0] = bs
        mxd_ref[0, 0] = bd

    @pl.when(i > 0)
    def _():
        mxs_ref[0, 0] = jnp.maximum(mxs_ref[0, 0], bs)
        mxd_ref[0, 0] = jnp.maximum(mxd_ref[0, 0], bd)


def _edge_body(er_ref, We1_ref, ae1_ref, We2_ref, ae2_ref,
               ae81_ref, ae82_ref, mxe1_ref, mxe2_ref):
    i = pl.program_id(0)
    er = er_ref[...]
    rowg = lax.broadcasted_iota(jnp.int32, (128, 128), 0) // 16
    colg = lax.broadcasted_iota(jnp.int32, (128, 128), 1) % 8
    mask = (rowg == colg).astype(jnp.float32)
    for We_ref, ae_ref, out_ref, mx_ref in (
            (We1_ref, ae1_ref, ae81_ref, mxe1_ref),
            (We2_ref, ae2_ref, ae82_ref, mxe2_ref)):
        ve = jnp.sum(We_ref[...] * ae_ref[...], axis=1, keepdims=True)
        ve_rep = jnp.concatenate([ve] * 8, axis=0)
        M = ve_rep * mask
        full = jnp.dot(er, M, preferred_element_type=jnp.float32)
        out_ref[...] = full[:, :8]
        bm = jnp.max(full[:, :8])

        @pl.when(i == 0)
        def _():
            mx_ref[0, 0] = bm

        @pl.when(i > 0)
        def _():
            mx_ref[0, 0] = jnp.maximum(mx_ref[0, 0], bm)


def _final_body(acc_ref, den_ref, b_ref, out_ref):
    acc = acc_ref[0] + acc_ref[1]
    den = den_ref[0, :, 0:1] + den_ref[1, :, 0:1]
    out_ref[...] = acc / (den + 1e-16) + b_ref[...]


def _sca_body(NCH, srcb, dstb, aeb, asrcb, adstb, mxb, exb,
              src_v, dst_v, ae_v, asrc_v, adst_v, ex_v, mx_v):
    c = lax.axis_index("c")
    s = lax.axis_index("s")
    w = c * NS + s

    pltpu.sync_copy(srcb.at[pl.ds(w * NCH, NCH)], src_v)
    pltpu.sync_copy(dstb.at[pl.ds(w * NCH, NCH)], dst_v)
    pltpu.sync_copy(aeb.at[pl.ds(w * NCH, NCH)], ae_v)
    pltpu.sync_copy(asrcb, asrc_v)
    pltpu.sync_copy(adstb, adst_v)
    pltpu.sync_copy(mxb, mx_v)

    i0 = jnp.zeros((LANES,), jnp.int32)
    m = (plsc.load_gather(mx_v, [i0]) +
         plsc.load_gather(mx_v, [i0 + 1]) +
         plsc.load_gather(mx_v, [i0 + 2]))
    Cv = jnp.maximum(m, 0.2 * m)

    def _scal(ci, carry):
        for j in range(CH // 16):
            sv = src_v[ci, pl.ds(j * 16, 16)]
            dv = dst_v[ci, pl.ds(j * 16, 16)]
            av = ae_v[ci, pl.ds(j * 16, 16)]
            a = plsc.load_gather(asrc_v, [sv]) + plsc.load_gather(adst_v, [dv]) + av
            a = jnp.maximum(a, 0.2 * a)
            ex_v[ci, pl.ds(j * 16, 16)] = jnp.exp(a - Cv)
        return carry
    lax.fori_loop(0, NCH, _scal, 0)
    pltpu.sync_copy(ex_v, exb.at[pl.ds(w * NCH, NCH)])


def _scb_body(NCH, RPT, SUP, NP, xs_hbm, srcb, dstb, exb,
              accp, denp, sidx, didx, exc, iidx, exr_v,
              rows0, accS, denS):
    c = lax.axis_index("c")
    s = lax.axis_index("s")
    w = c * NS + s

    base = s * RPT
    npieces = -(-RPT // CH)

    lane = lax.iota(jnp.int32, LANES)
    for q in range(npieces):
        for j in range(CH // 16):
            v = jnp.minimum(jnp.full((LANES,), q * CH + j * 16, jnp.int32) + lane,
                            RPT - 1)
            iidx[q, pl.ds(j * 16, 16)] = v + base

    zero16 = jnp.zeros((LANES,), jnp.float32)

    def _zr(k, carry):
        for j in range(8):
            rows0[k, pl.ds(j * 16, 16)] = zero16
        exr_v[k, :] = zero16
        return carry
    lax.fori_loop(0, CH, _zr, 0)
    for q in range(npieces):
        pltpu.sync_copy(rows0, accS.at[iidx.at[q]])
        pltpu.sync_copy(exr_v, denS.at[iidx.at[q]])
    plsc.subcore_barrier()

    NSUP = NCH // SUP

    def _work(q, rv):
        def _scale(k, c2):
            bc = plsc.load_gather(
                exc, [jnp.full((LANES,), q, jnp.int32),
                      jnp.full((LANES,), k, jnp.int32)])
            for j in range(8):
                rv[k, pl.ds(j * 16, 16)] = rv[k, pl.ds(j * 16, 16)] * bc
            exr_v[k, :] = bc
            return c2
        lax.fori_loop(0, CH, _scale, 0)
        pltpu.sync_copy(rv, accS.at[didx.at[q]], add=True)
        pltpu.sync_copy(exr_v, denS.at[didx.at[q]], add=True)

    def _sup(u, carry):
        b0 = w * NCH + u * SUP
        pltpu.sync_copy(srcb.at[pl.ds(b0, SUP)], sidx)
        pltpu.sync_copy(dstb.at[pl.ds(b0, SUP)], didx)
        pltpu.sync_copy(exb.at[pl.ds(b0, SUP)], exc)
        for q in range(SUP):
            pltpu.sync_copy(xs_hbm.at[sidx.at[q]], rows0)
            _work(q, rows0)
        return carry
    lax.fori_loop(0, NSUP, _sup, 0)

    plsc.subcore_barrier()

    obase = c * NP + base
    for q in range(npieces):
        p = min(CH, RPT - q * CH)
        pltpu.sync_copy(accS.at[iidx.at[q]], rows0)
        pltpu.sync_copy(denS.at[iidx.at[q]], exr_v)
        pltpu.sync_copy(rows0.at[pl.ds(0, p)], accp.at[pl.ds(obase + q * CH, p)])
        pltpu.sync_copy(exr_v.at[pl.ds(0, p)], denp.at[pl.ds(obase + q * CH, p)])


def _dense_call(body, n_extra_in, BN, N, D, H):
    grid = (N // BN,)
    smem_spec = pl.BlockSpec((1, 1), lambda i: (0, 0), memory_space=pltpu.SMEM)
    full = lambda shape: pl.BlockSpec(shape, lambda i: (0,) * len(shape))
    in_specs = (n_extra_in
                + [pl.BlockSpec((BN, D), lambda i: (i, 0)),
                   full((D, H)), full((D, H)), full((1, H)), full((1, H))])
    return pl.pallas_call(
        body,
        grid=grid,
        in_specs=in_specs,
        out_specs=[pl.BlockSpec((BN, H), lambda i: (i, 0)),
                   pl.BlockSpec((BN, 1), lambda i: (i, 0)),
                   pl.BlockSpec((BN, 1), lambda i: (i, 0)),
                   smem_spec, smem_spec],
        out_shape=[jax.ShapeDtypeStruct((N, H), jnp.float32),
                   jax.ShapeDtypeStruct((N, 1), jnp.float32),
                   jax.ShapeDtypeStruct((N, 1), jnp.float32),
                   jax.ShapeDtypeStruct((1, 1), jnp.float32),
                   jax.ShapeDtypeStruct((1, 1), jnp.float32)],
    )


def kernel(x, edge_index, edge_attr,
           Ws1, Wd1, We1, as1, ad1, ae1, b1,
           Ws2, Wd2, We2, as2, ad2, ae2, b2):
    N, D = x.shape
    H = Ws1.shape[1]
    OUT = Ws2.shape[1]
    E = edge_index.shape[1]
    DE = edge_attr.shape[1]

    W = NC * NS
    NCH = -(-E // (W * CH))
    NCH = -(-NCH // 8) * 8
    EP = W * NCH * CH
    pad = EP - E
    RPT = -(-N // NS)
    RPT = -(-RPT // 8) * 8
    NP = NS * RPT
    xp = jnp.concatenate([x, jnp.zeros((NP - N, D), jnp.float32)])

    src = edge_index[0].astype(jnp.int32)
    dst = edge_index[1].astype(jnp.int32)
    zpad = jnp.zeros((pad,), jnp.int32)
    srcb = jnp.concatenate([src, zpad]).reshape(W * NCH, CH)
    dstb = jnp.concatenate([dst, zpad]).reshape(W * NCH, CH)

    ER = E * DE // 128
    er = edge_attr.reshape(ER, 128)
    BE = 2000
    smem_spec = pl.BlockSpec((1, 1), lambda i: (0, 0), memory_space=pltpu.SMEM)
    full = lambda shape: pl.BlockSpec(shape, lambda i: (0,) * len(shape))
    ae81, ae82, mxe1, mxe2 = pl.pallas_call(
        _edge_body,
        grid=(ER // BE,),
        in_specs=[pl.BlockSpec((BE, 128), lambda i: (i, 0)),
                  full((DE, H)), full((1, H)), full((DE, OUT)), full((1, OUT))],
        out_specs=[pl.BlockSpec((BE, 8), lambda i: (i, 0)),
                   pl.BlockSpec((BE, 8), lambda i: (i, 0)),
                   smem_spec, smem_spec],
        out_shape=[jax.ShapeDtypeStruct((ER, 8), jnp.float32),
                   jax.ShapeDtypeStruct((ER, 8), jnp.float32),
                   jax.ShapeDtypeStruct((1, 1), jnp.float32),
                   jax.ShapeDtypeStruct((1, 1), jnp.float32)],
    )(er, We1, ae1.reshape(1, H), We2, ae2.reshape(1, OUT))
    apad = jnp.full((pad,), -1e30, jnp.float32)
    aeb1 = jnp.concatenate([ae81.reshape(E), apad]).reshape(W * NCH, CH)
    aeb2 = jnp.concatenate([ae82.reshape(E), apad]).reshape(W * NCH, CH)

    BN = RPT
    xs1, asrc1, adst1, mxs1, mxd1 = _dense_call(_dense_body, [], BN, NP, D, H)(
        xp, Ws1, Wd1, as1.reshape(1, H), ad1.reshape(1, H))
    mx1 = jnp.concatenate([mxs1.reshape(1), mxd1.reshape(1), mxe1.reshape(1),
                           jnp.zeros(13, jnp.float32)])

    SUP = 8
    mesh = plsc.VectorSubcoreMesh(core_axis_name="c", subcore_axis_name="s")
    sca_call = pl.kernel(
        functools.partial(_sca_body, NCH),
        out_type=jax.ShapeDtypeStruct((W * NCH, CH), jnp.float32),
        mesh=mesh,
        scratch_types=[
            pltpu.VMEM((NCH, CH), jnp.int32),
            pltpu.VMEM((NCH, CH), jnp.int32),
            pltpu.VMEM((NCH, CH), jnp.float32),
            pltpu.VMEM((NP,), jnp.float32),
            pltpu.VMEM((NP,), jnp.float32),
            pltpu.VMEM((NCH, CH), jnp.float32),
            pltpu.VMEM((16,), jnp.float32),
        ],
        compiler_params=pltpu.CompilerParams(needs_layout_passes=False),
    )
    scb_call = pl.kernel(
        functools.partial(_scb_body, NCH, RPT, SUP, NP),
        out_type=[jax.ShapeDtypeStruct((NC * NP, H), jnp.float32),
                  jax.ShapeDtypeStruct((NC * NP, 16), jnp.float32)],
        mesh=mesh,
        scratch_types=[
            pltpu.VMEM((SUP, CH), jnp.int32),
            pltpu.VMEM((SUP, CH), jnp.int32),
            pltpu.VMEM((SUP, CH), jnp.float32),
            pltpu.VMEM((-(-RPT // CH), CH), jnp.int32),
            pltpu.VMEM((CH, 16), jnp.float32),
            pltpu.VMEM((CH, H), jnp.float32),
            pltpu.VMEM_SHARED((NP, H), jnp.float32),
            pltpu.VMEM_SHARED((NP, 16), jnp.float32),
        ],
        compiler_params=pltpu.CompilerParams(needs_layout_passes=False),
    )

    exb1 = sca_call(srcb, dstb, aeb1, asrc1.reshape(NP), adst1.reshape(NP), mx1)
    acc1, den1 = scb_call(xs1, srcb, dstb, exb1)
    acc1 = acc1.reshape(NC, NP, H)
    den1 = den1.reshape(NC, NP, 16)

    xs2, asrc2, adst2, mxs2, mxd2 = pl.pallas_call(
        _fused_body,
        grid=(NP // BN,),
        in_specs=[pl.BlockSpec((NC, BN, H), lambda i: (0, i, 0)),
                  pl.BlockSpec((NC, BN, 16), lambda i: (0, i, 0)),
                  full((1, H)), full((H, OUT)), full((H, OUT)),
                  full((1, OUT)), full((1, OUT))],
        out_specs=[pl.BlockSpec((BN, OUT), lambda i: (i, 0)),
                   pl.BlockSpec((BN, 1), lambda i: (i, 0)),
                   pl.BlockSpec((BN, 1), lambda i: (i, 0)),
                   smem_spec, smem_spec],
        out_shape=[jax.ShapeDtypeStruct((NP, OUT), jnp.float32),
                   jax.ShapeDtypeStruct((NP, 1), jnp.float32),
                   jax.ShapeDtypeStruct((NP, 1), jnp.float32),
                   jax.ShapeDtypeStruct((1, 1), jnp.float32),
                   jax.ShapeDtypeStruct((1, 1), jnp.float32)],
    )(acc1, den1, b1.reshape(1, H), Ws2, Wd2,
      as2.reshape(1, OUT), ad2.reshape(1, OUT))
    mx2 = jnp.concatenate([mxs2.reshape(1), mxd2.reshape(1), mxe2.reshape(1),
                           jnp.zeros(13, jnp.float32)])

    exb2 = sca_call(srcb, dstb, aeb2, asrc2.reshape(NP), adst2.reshape(NP), mx2)
    acc2, den2 = scb_call(xs2, srcb, dstb, exb2)
    acc2 = acc2.reshape(NC, NP, H)
    den2 = den2.reshape(NC, NP, 16)

    out = pl.pallas_call(
        _final_body,
        grid=(NP // BN,),
        in_specs=[pl.BlockSpec((NC, BN, OUT), lambda i: (0, i, 0)),
                  pl.BlockSpec((NC, BN, 16), lambda i: (0, i, 0)),
                  full((1, OUT))],
        out_specs=pl.BlockSpec((BN, OUT), lambda i: (i, 0)),
        out_shape=jax.ShapeDtypeStruct((NP, OUT), jnp.float32),
    )(acc2, den2, b2.reshape(1, OUT))
    return out[:N]
